# baseline (speedup 1.0000x reference)
import functools

import jax
import jax.numpy as jnp
from jax import lax
from jax.experimental import pallas as pl
from jax.experimental.pallas import tpu as pltpu
from jax.experimental.pallas import tpu_sc as plsc

N, K, C = 10000, 32, 128
HID = 64
EH = 32
D = 128
E = N * K


def _pack_body(x_ref, boxes_ref, wy_ref, wak_ref, o_ref):
    xb = x_ref[...]
    y = jnp.dot(xb, wy_ref[...], preferred_element_type=jnp.float32)
    ak = jnp.dot(xb, wak_ref[...], preferred_element_type=jnp.float32)
    pad = jnp.zeros((xb.shape[0], D - (HID + 1 + 2 + 5)), jnp.float32)
    o_ref[...] = jnp.concatenate(
        [y, ak, xb[:, 0:2], boxes_ref[...], pad], axis=1)


def _pack_table(x, boxes, wy, wak, *, interpret=False):
    blk = 2000
    return pl.pallas_call(
        _pack_body,
        grid=(N // blk,),
        in_specs=[
            pl.BlockSpec((blk, C), lambda i: (i, 0)),
            pl.BlockSpec((blk, 5), lambda i: (i, 0)),
            pl.BlockSpec((C, HID), lambda i: (0, 0)),
            pl.BlockSpec((C, 1), lambda i: (0, 0)),
        ],
        out_specs=pl.BlockSpec((blk, D), lambda i: (i, 0)),
        out_shape=jax.ShapeDtypeStruct((N, D), jnp.float32),
        interpret=interpret,
    )(x, boxes, wy, wak)


_CH = 80
_PER_W = E // 32
_NCHUNK = _PER_W // _CH


def _sc_gather(table, idx):
    mesh = plsc.VectorSubcoreMesh(core_axis_name="c", subcore_axis_name="s")

    @functools.partial(
        pl.kernel,
        mesh=mesh,
        out_type=jax.ShapeDtypeStruct((E, D), jnp.float32),
        scratch_types=[
            pltpu.VMEM((_CH,), jnp.int32),
            pltpu.VMEM((_CH, D), jnp.float32),
            pltpu.SemaphoreType.DMA,
        ],
        compiler_params=pltpu.CompilerParams(use_tc_tiling_on_sc=False),
    )
    def gk(t_hbm, i_hbm, o_hbm, idx_v, rows_v, sem):
        wid = lax.axis_index("s") * 2 + lax.axis_index("c")
        base = pl.multiple_of(wid * _PER_W, 8)

        def body(ci, _):
            off = pl.multiple_of(base + ci * _CH, 8)
            pltpu.sync_copy(i_hbm.at[pl.ds(off, _CH)], idx_v)
            pltpu.async_copy(t_hbm.at[idx_v], rows_v, sem).wait()
            pltpu.sync_copy(rows_v, o_hbm.at[pl.ds(off, _CH)])
            return 0

        lax.fori_loop(0, _NCHUNK, body, 0)

    return gk(table, idx)


def _msg_body(g_ref, x_ref, boxes_ref, we1_ref, be1_ref, we2_ref, be2_ref,
              wh_ref, rot_ref, waq_ref, bm1_ref, wm2_ref, bm2_ref, o_ref):
    blk = x_ref.shape[0]
    bk = blk * K

    g = g_ref[...]
    y_g = g[:, 0:HID]
    ak_g = g[:, HID:HID + 1]
    x0n = g[:, 65:66]
    x1n = g[:, 66:67]
    bnx = g[:, 67:68]
    bny = g[:, 68:69]
    bnw = g[:, 69:70]
    bnh = g[:, 70:71]
    bnt = g[:, 71:72]

    boxes_i = boxes_ref[...]
    def ecol(col):
        return jnp.broadcast_to(
            col.reshape(blk, 1, 1), (blk, K, 1)).reshape(bk, 1)

    bix = ecol(boxes_i[:, 0:1])
    biy = ecol(boxes_i[:, 1:2])
    biw = ecol(boxes_i[:, 2:3])
    bih = ecol(boxes_i[:, 3:4])
    bit = ecol(boxes_i[:, 4:5])

    dth = bit - bnt
    cth = jnp.cos(dth)
    sth = jnp.sin(dth)
    dx = bix - bnx
    dy = biy - bny
    dist = jnp.sqrt(dx * dx + dy * dy + 1e-12)
    ratio = jnp.minimum(biw, bih) / jnp.minimum(bnw, bnh)
    scale = jnp.log(jnp.maximum(ratio, 1e-6))

    we1 = we1_ref[...]
    h1 = jax.nn.relu(dist * we1[0:1, :] + scale * we1[1:2, :]
                     + cth * we1[2:3, :] + sth * we1[3:4, :] + be1_ref[...])
    e_emb = jax.nn.relu(
        jnp.dot(h1, we2_ref[...], preferred_element_type=jnp.float32)
        + be2_ref[...])

    u0 = (cth - 1.0) * x0n - sth * x1n
    u1 = sth * x0n + (cth - 1.0) * x1n

    m = jnp.dot(e_emb, wh_ref[...], preferred_element_type=jnp.float32)
    rot = rot_ref[...]
    m = m + u0 * rot[0:1, :] + u1 * rot[1:2, :]

    pre_h = y_g + m[:, 0:HID] + bm1_ref[...]
    h = jax.nn.relu(pre_h)

    a_q = jnp.dot(x_ref[...], waq_ref[...],
                  preferred_element_type=jnp.float32)
    logits = ak_g + m[:, HID:HID + 1] + ecol(a_q)

    l3 = logits.reshape(blk, K, 1)
    lmax = jnp.max(l3, axis=1, keepdims=True)
    p = jnp.exp(l3 - lmax)
    alpha3 = p / jnp.sum(p, axis=1, keepdims=True)
    alpha = alpha3.reshape(bk, 1)

    agg = jnp.sum((alpha * h).reshape(blk, K, HID), axis=1)
    msg = jnp.dot(agg, wm2_ref[...],
                  preferred_element_type=jnp.float32) + bm2_ref[...]
    o_ref[...] = x_ref[...] + msg


def _message(g, x, boxes, we1, be1, we2, be2, wh, rot, waq, bm1, wm2, bm2,
             *, interpret=False):
    blk = 400
    bk = blk * K
    return pl.pallas_call(
        _msg_body,
        grid=(N // blk,),
        in_specs=[
            pl.BlockSpec((bk, D), lambda i: (i, 0)),
            pl.BlockSpec((blk, C), lambda i: (i, 0)),
            pl.BlockSpec((blk, 5), lambda i: (i, 0)),
            pl.BlockSpec((4, EH), lambda i: (0, 0)),
            pl.BlockSpec((1, EH), lambda i: (0, 0)),
            pl.BlockSpec((EH, EH), lambda i: (0, 0)),
            pl.BlockSpec((1, EH), lambda i: (0, 0)),
            pl.BlockSpec((EH, HID + 1), lambda i: (0, 0)),
            pl.BlockSpec((2, HID + 1), lambda i: (0, 0)),
            pl.BlockSpec((C, 1), lambda i: (0, 0)),
            pl.BlockSpec((1, HID), lambda i: (0, 0)),
            pl.BlockSpec((HID, C), lambda i: (0, 0)),
            pl.BlockSpec((1, C), lambda i: (0, 0)),
        ],
        out_specs=pl.BlockSpec((blk, C), lambda i: (i, 0)),
        out_shape=jax.ShapeDtypeStruct((N, C), jnp.float32),
        compiler_params=pltpu.CompilerParams(
            dimension_semantics=("arbitrary",)),
        interpret=interpret,
    )(g, x, boxes, we1, be1, we2, be2, wh, rot, waq, bm1, wm2, bm2)


def kernel(x, nbr_idx, boxes, W_e1, b_e1, W_e2, b_e2, W_m1, b_m1, W_m2, b_m2,
           W_a, b_a):
    x = x.astype(jnp.float32)
    idx = nbr_idx.reshape(-1).astype(jnp.int32)

    wy = W_m1[:C, :]
    wak = W_a[C:2 * C, :]
    table = _pack_table(x, boxes, wy, wak)

    g = _sc_gather(table, idx)

    wh = jnp.concatenate([W_m1[C:, :], W_a[2 * C:, :]], axis=1)
    rot = jnp.concatenate(
        [W_m1[0:2, :], W_a[C:C + 2, :]], axis=1)
    waq = W_a[:C, :]
    out = _message(g, x, boxes,
                   W_e1, b_e1.reshape(1, EH), W_e2, b_e2.reshape(1, EH),
                   wh, rot, waq, b_m1.reshape(1, HID), W_m2,
                   b_m2.reshape(1, C))
    return out


# speedup vs baseline: 2.4503x; 2.4503x over previous
import functools

import jax
import jax.numpy as jnp
from jax import lax
from jax.experimental import pallas as pl
from jax.experimental.pallas import tpu as pltpu
from jax.experimental.pallas import tpu_sc as plsc

N, K, C = 10000, 32, 128
HID = 64
EH = 32
DY = 64
DS = 16
E = N * K
_LOG1EM6 = -13.815510557964274


def _pack_body(x_ref, boxes_ref, wy_ref, wak_ref, oy_ref, os_ref):
    xb = x_ref[...]
    b = boxes_ref[...]
    oy_ref[...] = jnp.dot(xb, wy_ref[...], preferred_element_type=jnp.float32)
    ak = jnp.dot(xb, wak_ref[...], preferred_element_type=jnp.float32)
    th = b[:, 4:5]
    lm = jnp.log(jnp.minimum(b[:, 2:3], b[:, 3:4]))
    pad = jnp.zeros((xb.shape[0], DS - 8), jnp.float32)
    os_ref[...] = jnp.concatenate(
        [ak, xb[:, 0:2], b[:, 0:2], jnp.cos(th), jnp.sin(th), lm, pad],
        axis=1)


def _pack_tables(x, boxes, wy, wak, *, interpret=False):
    blk = 2000
    return pl.pallas_call(
        _pack_body,
        grid=(N // blk,),
        in_specs=[
            pl.BlockSpec((blk, C), lambda i: (i, 0)),
            pl.BlockSpec((blk, 5), lambda i: (i, 0)),
            pl.BlockSpec((C, HID), lambda i: (0, 0)),
            pl.BlockSpec((C, 1), lambda i: (0, 0)),
        ],
        out_specs=[
            pl.BlockSpec((blk, DY), lambda i: (i, 0)),
            pl.BlockSpec((blk, DS), lambda i: (i, 0)),
        ],
        out_shape=[
            jax.ShapeDtypeStruct((N, DY), jnp.float32),
            jax.ShapeDtypeStruct((N, DS), jnp.float32),
        ],
        interpret=interpret,
    )(x, boxes, wy, wak)


_CH = 80
_PER_W = E // 32
_NCHUNK = _PER_W // _CH


def _sc_gather(ty, ts, idx):
    mesh = plsc.VectorSubcoreMesh(core_axis_name="c", subcore_axis_name="s")

    @functools.partial(
        pl.kernel,
        mesh=mesh,
        out_type=(
            jax.ShapeDtypeStruct((E, DY), jnp.float32),
            jax.ShapeDtypeStruct((8, E), jnp.float32),
        ),
        scratch_types=[
            pltpu.VMEM((_CH,), jnp.int32),
            pltpu.VMEM((_CH, DY), jnp.float32),
            pltpu.VMEM((_CH, DS), jnp.float32),
            pltpu.VMEM((8, _CH), jnp.float32),
            pltpu.SemaphoreType.DMA,
            pltpu.SemaphoreType.DMA,
        ],
        compiler_params=pltpu.CompilerParams(use_tc_tiling_on_sc=False,
                                             needs_layout_passes=False),
    )
    def gk(ty_hbm, ts_hbm, idx_hbm, gy_hbm, gs_hbm,
           idx_v, rowsy_v, rowss_v, planes_v, sem1, sem2):
        wid = lax.axis_index("s") * 2 + lax.axis_index("c")
        base = pl.multiple_of(wid * _PER_W, 8)

        def body(ci, _):
            off = pl.multiple_of(base + ci * _CH, 8)
            pltpu.sync_copy(idx_hbm.at[pl.ds(off, _CH)], idx_v)
            cpy = pltpu.async_copy(ty_hbm.at[idx_v], rowsy_v, sem1)
            cps = pltpu.async_copy(ts_hbm.at[idx_v], rowss_v, sem2)
            cps.wait()
            for c in range(8):
                cid = jnp.full((16,), c, jnp.int32)
                for g in range(_CH // 16):
                    rid = lax.iota(jnp.int32, 16) + (16 * g)
                    v = plsc.load_gather(rowss_v, [rid, cid])
                    planes_v[c, pl.ds(16 * g, 16)] = v
            for c in range(8):
                pltpu.sync_copy(planes_v.at[c], gs_hbm.at[c, pl.ds(off, _CH)])
            cpy.wait()
            pltpu.sync_copy(rowsy_v, gy_hbm.at[pl.ds(off, _CH)])
            return 0

        lax.fori_loop(0, _NCHUNK, body, 0)

    return gk(ty, ts, idx)


def _msg_body(gy_ref, gs_ref, boxes_ref, x_ref, we1t_ref, be1_ref, we2t_ref,
              be2_ref, w34_ref, wal_ref, bm1_ref, wm2_ref, bm2_ref, o_ref):
    blk = x_ref.shape[0]
    bk = blk * K

    gs = gs_ref[...]
    ak = gs[0:1, :]
    x0n = gs[1:2, :]
    x1n = gs[2:3, :]
    bnx = gs[3:4, :]
    bny = gs[4:5, :]
    cn = gs[5:6, :]
    sn = gs[6:7, :]
    lmn = gs[7:8, :]

    rowid = lax.broadcasted_iota(jnp.int32, (blk, bk), 0)
    colid = lax.broadcasted_iota(jnp.int32, (blk, bk), 1)
    seg2 = (colid // K == rowid).astype(jnp.float32)

    b5 = boxes_ref[...]
    th = b5[:, 4:5]
    d5 = jnp.concatenate(
        [b5[:, 0:2], jnp.cos(th), jnp.sin(th),
         jnp.log(jnp.minimum(b5[:, 2:3], b5[:, 3:4]))], axis=1)
    dp = lax.dot_general(d5, seg2, (((0,), (0,)), ((), ())),
                         preferred_element_type=jnp.float32)
    bix = dp[0:1, :]
    biy = dp[1:2, :]
    ci = dp[2:3, :]
    si = dp[3:4, :]
    lmi = dp[4:5, :]

    cth = ci * cn + si * sn
    sth = si * cn - ci * sn
    dx = bix - bnx
    dy = biy - bny
    dist = jnp.sqrt(dx * dx + dy * dy + 1e-12)
    scale = jnp.maximum(lmi - lmn, _LOG1EM6)
    u0 = (cth - 1.0) * x0n - sth * x1n
    u1 = sth * x0n + (cth - 1.0) * x1n

    e4 = jnp.concatenate([dist, scale, cth, sth], axis=0)
    h1t = jax.nn.relu(
        jnp.dot(we1t_ref[...], e4, preferred_element_type=jnp.float32)
        + be1_ref[...])
    e_embt = jax.nn.relu(
        jnp.dot(we2t_ref[...], h1t, preferred_element_type=jnp.float32)
        + be2_ref[...])

    l34 = jnp.concatenate([e_embt, u0, u1], axis=0)
    logits = ak + jnp.dot(wal_ref[...], l34,
                          preferred_element_type=jnp.float32)
    p = jnp.exp(logits)
    seg_p = seg2 * p

    m_hid = lax.dot_general(l34, w34_ref[...], (((0,), (0,)), ((), ())),
                            preferred_element_type=jnp.float32)
    h = jax.nn.relu(gy_ref[...] + m_hid + bm1_ref[...])

    agg = jnp.dot(seg_p, h, preferred_element_type=jnp.float32)
    den = jnp.sum(seg_p, axis=1, keepdims=True)
    agg = agg / den
    msg = jnp.dot(agg, wm2_ref[...],
                  preferred_element_type=jnp.float32) + bm2_ref[...]
    o_ref[...] = x_ref[...] + msg


def _message(gy, gs, boxes, x, we1t, be1, we2t, be2, w34, wal, bm1, wm2, bm2,
             *, interpret=False):
    blk = 80
    bk = blk * K
    return pl.pallas_call(
        _msg_body,
        grid=(N // blk,),
        in_specs=[
            pl.BlockSpec((bk, DY), lambda i: (i, 0)),
            pl.BlockSpec((8, bk), lambda i: (0, i)),
            pl.BlockSpec((blk, 5), lambda i: (i, 0)),
            pl.BlockSpec((blk, C), lambda i: (i, 0)),
            pl.BlockSpec((EH, 4), lambda i: (0, 0)),
            pl.BlockSpec((EH, 1), lambda i: (0, 0)),
            pl.BlockSpec((EH, EH), lambda i: (0, 0)),
            pl.BlockSpec((EH, 1), lambda i: (0, 0)),
            pl.BlockSpec((EH + 2, HID), lambda i: (0, 0)),
            pl.BlockSpec((1, EH + 2), lambda i: (0, 0)),
            pl.BlockSpec((1, HID), lambda i: (0, 0)),
            pl.BlockSpec((HID, C), lambda i: (0, 0)),
            pl.BlockSpec((1, C), lambda i: (0, 0)),
        ],
        out_specs=pl.BlockSpec((blk, C), lambda i: (i, 0)),
        out_shape=jax.ShapeDtypeStruct((N, C), jnp.float32),
        compiler_params=pltpu.CompilerParams(
            dimension_semantics=("arbitrary",)),
        interpret=interpret,
    )(gy, gs, boxes, x, we1t, be1, we2t, be2, w34, wal, bm1, wm2, bm2)


def kernel(x, nbr_idx, boxes, W_e1, b_e1, W_e2, b_e2, W_m1, b_m1, W_m2, b_m2,
           W_a, b_a):
    x = x.astype(jnp.float32)
    idx = nbr_idx.reshape(-1).astype(jnp.int32)

    wy = W_m1[:C, :]
    wak = W_a[C:2 * C, :]
    ty, ts = _pack_tables(x, boxes, wy, wak)

    gy, gs = _sc_gather(ty, ts, idx)

    w34 = jnp.concatenate([W_m1[C:, :], W_m1[0:2, :]], axis=0)
    wal = jnp.concatenate([W_a[2 * C:, :], W_a[C:C + 2, :]],
                          axis=0).T
    out = _message(gy, gs, boxes, x,
                   W_e1.T, b_e1.reshape(EH, 1), W_e2.T, b_e2.reshape(EH, 1),
                   w34, wal, b_m1.reshape(1, HID), W_m2, b_m2.reshape(1, C))
    return out


# speedup vs baseline: 3.2776x; 1.3377x over previous
import functools

import jax
import jax.numpy as jnp
from jax import lax
from jax.experimental import pallas as pl
from jax.experimental.pallas import tpu as pltpu
from jax.experimental.pallas import tpu_sc as plsc

N, K, C = 10000, 32, 128
HID = 64
EH = 32
DY = 64
DS = 16
E = N * K
_LOG1EM6 = -13.815510557964274


def _pack_body(x_ref, boxes_ref, wy_ref, wak_ref, oy_ref, os_ref):
    xb = x_ref[...]
    b = boxes_ref[...]
    oy_ref[...] = jnp.dot(xb, wy_ref[...], preferred_element_type=jnp.float32)
    ak = jnp.dot(xb, wak_ref[...], preferred_element_type=jnp.float32)
    th = b[:, 4:5]
    lm = jnp.log(jnp.minimum(b[:, 2:3], b[:, 3:4]))
    pad = jnp.zeros((xb.shape[0], DS - 8), jnp.float32)
    os_ref[...] = jnp.concatenate(
        [ak, xb[:, 0:2], b[:, 0:2], jnp.cos(th), jnp.sin(th), lm, pad],
        axis=1)


def _pack_tables(x, boxes, wy, wak, *, interpret=False):
    blk = 2000
    return pl.pallas_call(
        _pack_body,
        grid=(N // blk,),
        in_specs=[
            pl.BlockSpec((blk, C), lambda i: (i, 0)),
            pl.BlockSpec((blk, 5), lambda i: (i, 0)),
            pl.BlockSpec((C, HID), lambda i: (0, 0)),
            pl.BlockSpec((C, 1), lambda i: (0, 0)),
        ],
        out_specs=[
            pl.BlockSpec((blk, DY), lambda i: (i, 0)),
            pl.BlockSpec((blk, DS), lambda i: (i, 0)),
        ],
        out_shape=[
            jax.ShapeDtypeStruct((N, DY), jnp.float32),
            jax.ShapeDtypeStruct((N, DS), jnp.float32),
        ],
        interpret=interpret,
    )(x, boxes, wy, wak)


_CH = 400
_PER_W = E // 32
_NCHUNK = _PER_W // _CH


def _sc_gather(ty, ts, idx):
    mesh = plsc.VectorSubcoreMesh(core_axis_name="c", subcore_axis_name="s")

    @functools.partial(
        pl.kernel,
        mesh=mesh,
        out_type=(
            jax.ShapeDtypeStruct((E, DY), jnp.float32),
            jax.ShapeDtypeStruct((8, E), jnp.float32),
        ),
        scratch_types=[
            pltpu.VMEM((_PER_W,), jnp.int32),
            pltpu.VMEM((_CH, DY), jnp.float32),
            pltpu.VMEM((_CH, DY), jnp.float32),
            pltpu.VMEM((_CH, DS), jnp.float32),
            pltpu.VMEM((_CH, DS), jnp.float32),
            pltpu.VMEM((8, _CH), jnp.float32),
            pltpu.SemaphoreType.DMA,
            pltpu.SemaphoreType.DMA,
            pltpu.SemaphoreType.DMA,
        ],
        compiler_params=pltpu.CompilerParams(use_tc_tiling_on_sc=False,
                                             needs_layout_passes=False),
    )
    def gk(ty_hbm, ts_hbm, idx_hbm, gy_hbm, gs_hbm,
           idx_v, ya_v, yb_v, sa_v, sb_v, planes_v, syg, ssg, sw):
        wid = lax.axis_index("s") * 2 + lax.axis_index("c")
        base = pl.multiple_of(wid * _PER_W, 8)
        pltpu.sync_copy(idx_hbm.at[pl.ds(base, _PER_W)], idx_v)

        def start(ci, yv, sv):
            ix = idx_v.at[pl.ds(ci * _CH, _CH)]
            cy = pltpu.async_copy(ty_hbm.at[ix], yv, syg)
            cs = pltpu.async_copy(ts_hbm.at[ix], sv, ssg)
            return cy, cs

        def proc(ci, yv, sv, cy, cs):
            off = pl.multiple_of(base + ci * _CH, 8)
            cs.wait()
            for c in range(8):
                cid = jnp.full((16,), c, jnp.int32)
                for g in range(_CH // 16):
                    rid = lax.iota(jnp.int32, 16) + (16 * g)
                    planes_v[c, pl.ds(16 * g, 16)] = plsc.load_gather(
                        sv, [rid, cid])
            wr = [pltpu.async_copy(planes_v.at[c],
                                   gs_hbm.at[c, pl.ds(off, _CH)], sw)
                  for c in range(8)]
            cy.wait()
            wr.append(pltpu.async_copy(yv, gy_hbm.at[pl.ds(off, _CH)], sw))
            for h in wr:
                h.wait()

        def body(j, _):
            ca = start(2 * j, ya_v, sa_v)
            cb = start(2 * j + 1, yb_v, sb_v)
            proc(2 * j, ya_v, sa_v, *ca)
            proc(2 * j + 1, yb_v, sb_v, *cb)
            return 0

        lax.fori_loop(0, _NCHUNK // 2, body, 0)
        proc(_NCHUNK - 1, ya_v, sa_v, *start(_NCHUNK - 1, ya_v, sa_v))

    return gk(ty, ts, idx)


def _msg_body(gy_ref, gs_ref, boxes_ref, x_ref, we1t_ref, be1_ref, we2t_ref,
              be2_ref, w34_ref, wal_ref, bm1_ref, wm2_ref, bm2_ref, o_ref):
    blk = x_ref.shape[0]
    bk = blk * K

    gs = gs_ref[...]
    ak = gs[0:1, :]
    x0n = gs[1:2, :]
    x1n = gs[2:3, :]
    bnx = gs[3:4, :]
    bny = gs[4:5, :]
    cn = gs[5:6, :]
    sn = gs[6:7, :]
    lmn = gs[7:8, :]

    rowid = lax.broadcasted_iota(jnp.int32, (blk, bk), 0)
    colid = lax.broadcasted_iota(jnp.int32, (blk, bk), 1)
    seg2 = (colid // K == rowid).astype(jnp.float32)

    b5 = boxes_ref[...]
    th = b5[:, 4:5]
    d5 = jnp.concatenate(
        [b5[:, 0:2], jnp.cos(th), jnp.sin(th),
         jnp.log(jnp.minimum(b5[:, 2:3], b5[:, 3:4]))], axis=1)
    dp = lax.dot_general(d5, seg2, (((0,), (0,)), ((), ())),
                         preferred_element_type=jnp.float32)
    bix = dp[0:1, :]
    biy = dp[1:2, :]
    ci = dp[2:3, :]
    si = dp[3:4, :]
    lmi = dp[4:5, :]

    cth = ci * cn + si * sn
    sth = si * cn - ci * sn
    dx = bix - bnx
    dy = biy - bny
    dist = jnp.sqrt(dx * dx + dy * dy + 1e-12)
    scale = jnp.maximum(lmi - lmn, _LOG1EM6)
    u0 = (cth - 1.0) * x0n - sth * x1n
    u1 = sth * x0n + (cth - 1.0) * x1n

    e4 = jnp.concatenate([dist, scale, cth, sth], axis=0)
    h1t = jax.nn.relu(
        jnp.dot(we1t_ref[...], e4, preferred_element_type=jnp.float32)
        + be1_ref[...])
    e_embt = jax.nn.relu(
        jnp.dot(we2t_ref[...], h1t, preferred_element_type=jnp.float32)
        + be2_ref[...])

    l34 = jnp.concatenate([e_embt, u0, u1], axis=0)
    logits = ak + jnp.dot(wal_ref[...], l34,
                          preferred_element_type=jnp.float32)
    p = jnp.exp(logits)
    seg_p = seg2 * p

    m_hid = lax.dot_general(l34, w34_ref[...], (((0,), (0,)), ((), ())),
                            preferred_element_type=jnp.float32)
    h = jax.nn.relu(gy_ref[...] + m_hid + bm1_ref[...])

    agg = jnp.dot(seg_p, h, preferred_element_type=jnp.float32)
    den = jnp.sum(seg_p, axis=1, keepdims=True)
    agg = agg / den
    msg = jnp.dot(agg, wm2_ref[...],
                  preferred_element_type=jnp.float32) + bm2_ref[...]
    o_ref[...] = x_ref[...] + msg


def _message(gy, gs, boxes, x, we1t, be1, we2t, be2, w34, wal, bm1, wm2, bm2,
             *, interpret=False):
    blk = 80
    bk = blk * K
    return pl.pallas_call(
        _msg_body,
        grid=(N // blk,),
        in_specs=[
            pl.BlockSpec((bk, DY), lambda i: (i, 0)),
            pl.BlockSpec((8, bk), lambda i: (0, i)),
            pl.BlockSpec((blk, 5), lambda i: (i, 0)),
            pl.BlockSpec((blk, C), lambda i: (i, 0)),
            pl.BlockSpec((EH, 4), lambda i: (0, 0)),
            pl.BlockSpec((EH, 1), lambda i: (0, 0)),
            pl.BlockSpec((EH, EH), lambda i: (0, 0)),
            pl.BlockSpec((EH, 1), lambda i: (0, 0)),
            pl.BlockSpec((EH + 2, HID), lambda i: (0, 0)),
            pl.BlockSpec((1, EH + 2), lambda i: (0, 0)),
            pl.BlockSpec((1, HID), lambda i: (0, 0)),
            pl.BlockSpec((HID, C), lambda i: (0, 0)),
            pl.BlockSpec((1, C), lambda i: (0, 0)),
        ],
        out_specs=pl.BlockSpec((blk, C), lambda i: (i, 0)),
        out_shape=jax.ShapeDtypeStruct((N, C), jnp.float32),
        compiler_params=pltpu.CompilerParams(
            dimension_semantics=("arbitrary",)),
        interpret=interpret,
    )(gy, gs, boxes, x, we1t, be1, we2t, be2, w34, wal, bm1, wm2, bm2)


def kernel(x, nbr_idx, boxes, W_e1, b_e1, W_e2, b_e2, W_m1, b_m1, W_m2, b_m2,
           W_a, b_a):
    x = x.astype(jnp.float32)
    idx = nbr_idx.reshape(-1).astype(jnp.int32)

    wy = W_m1[:C, :]
    wak = W_a[C:2 * C, :]
    ty, ts = _pack_tables(x, boxes, wy, wak)

    gy, gs = _sc_gather(ty, ts, idx)

    w34 = jnp.concatenate([W_m1[C:, :], W_m1[0:2, :]], axis=0)
    wal = jnp.concatenate([W_a[2 * C:, :], W_a[C:C + 2, :]],
                          axis=0).T
    out = _message(gy, gs, boxes, x,
                   W_e1.T, b_e1.reshape(EH, 1), W_e2.T, b_e2.reshape(EH, 1),
                   w34, wal, b_m1.reshape(1, HID), W_m2, b_m2.reshape(1, C))
    return out


# speedup vs baseline: 3.2830x; 1.0016x over previous
import functools

import jax
import jax.numpy as jnp
from jax import lax
from jax.experimental import pallas as pl
from jax.experimental.pallas import tpu as pltpu
from jax.experimental.pallas import tpu_sc as plsc

N, K, C = 10000, 32, 128
HID = 64
EH = 32
DY = 64
DS = 16
E = N * K
_LOG1EM6 = -13.815510557964274


def _pack_body(x_ref, boxes_ref, wy_ref, wak_ref, oy_ref, os_ref):
    xb = x_ref[...]
    b = boxes_ref[...]
    oy_ref[...] = jnp.dot(xb, wy_ref[...], preferred_element_type=jnp.float32)
    ak = jnp.dot(xb, wak_ref[...], preferred_element_type=jnp.float32)
    th = b[:, 4:5]
    lm = jnp.log(jnp.minimum(b[:, 2:3], b[:, 3:4]))
    pad = jnp.zeros((xb.shape[0], DS - 8), jnp.float32)
    os_ref[...] = jnp.concatenate(
        [ak, xb[:, 0:2], b[:, 0:2], jnp.cos(th), jnp.sin(th), lm, pad],
        axis=1)


def _pack_tables(x, boxes, wy, wak, *, interpret=False):
    blk = 2000
    return pl.pallas_call(
        _pack_body,
        grid=(N // blk,),
        in_specs=[
            pl.BlockSpec((blk, C), lambda i: (i, 0)),
            pl.BlockSpec((blk, 5), lambda i: (i, 0)),
            pl.BlockSpec((C, HID), lambda i: (0, 0)),
            pl.BlockSpec((C, 1), lambda i: (0, 0)),
        ],
        out_specs=[
            pl.BlockSpec((blk, DY), lambda i: (i, 0)),
            pl.BlockSpec((blk, DS), lambda i: (i, 0)),
        ],
        out_shape=[
            jax.ShapeDtypeStruct((N, DY), jnp.float32),
            jax.ShapeDtypeStruct((N, DS), jnp.float32),
        ],
        interpret=interpret,
    )(x, boxes, wy, wak)


_CH = 400
_PER_W = E // 32
_NCHUNK = _PER_W // _CH


def _sc_gather(ty, ts, idx):
    mesh = plsc.VectorSubcoreMesh(core_axis_name="c", subcore_axis_name="s")

    @functools.partial(
        pl.kernel,
        mesh=mesh,
        out_type=(
            jax.ShapeDtypeStruct((E, DY), jnp.float32),
            jax.ShapeDtypeStruct((8, E), jnp.float32),
        ),
        scratch_types=[
            pltpu.VMEM((_PER_W,), jnp.int32),
            pltpu.VMEM((_CH, DY), jnp.float32),
            pltpu.VMEM((_CH, DY), jnp.float32),
            pltpu.VMEM((_CH, DS), jnp.float32),
            pltpu.VMEM((_CH, DS), jnp.float32),
            pltpu.VMEM((8, _CH), jnp.float32),
            pltpu.SemaphoreType.DMA,
            pltpu.SemaphoreType.DMA,
            pltpu.SemaphoreType.DMA,
        ],
        compiler_params=pltpu.CompilerParams(use_tc_tiling_on_sc=False,
                                             needs_layout_passes=False),
    )
    def gk(ty_hbm, ts_hbm, idx_hbm, gy_hbm, gs_hbm,
           idx_v, ya_v, yb_v, sa_v, sb_v, planes_v, syg, ssg, sw):
        wid = lax.axis_index("s") * 2 + lax.axis_index("c")
        base = pl.multiple_of(wid * _PER_W, 8)
        pltpu.sync_copy(idx_hbm.at[pl.ds(base, _PER_W)], idx_v)

        def start(ci, yv, sv):
            ix = idx_v.at[pl.ds(ci * _CH, _CH)]
            cy = pltpu.async_copy(ty_hbm.at[ix], yv, syg)
            cs = pltpu.async_copy(ts_hbm.at[ix], sv, ssg)
            return cy, cs

        def proc(ci, yv, sv, cy, cs):
            off = pl.multiple_of(base + ci * _CH, 8)
            cs.wait()
            for c in range(8):
                cid = jnp.full((16,), c, jnp.int32)
                for g in range(_CH // 16):
                    rid = lax.iota(jnp.int32, 16) + (16 * g)
                    planes_v[c, pl.ds(16 * g, 16)] = plsc.load_gather(
                        sv, [rid, cid])
            wr = [pltpu.async_copy(planes_v.at[c],
                                   gs_hbm.at[c, pl.ds(off, _CH)], sw)
                  for c in range(8)]
            cy.wait()
            wr.append(pltpu.async_copy(yv, gy_hbm.at[pl.ds(off, _CH)], sw))
            for h in wr:
                h.wait()

        def body(j, _):
            ca = start(2 * j, ya_v, sa_v)
            cb = start(2 * j + 1, yb_v, sb_v)
            proc(2 * j, ya_v, sa_v, *ca)
            proc(2 * j + 1, yb_v, sb_v, *cb)
            return 0

        lax.fori_loop(0, _NCHUNK // 2, body, 0)
        proc(_NCHUNK - 1, ya_v, sa_v, *start(_NCHUNK - 1, ya_v, sa_v))

    return gk(ty, ts, idx)


def _msg_body(gy_ref, gs_ref, boxes_ref, x_ref, seg2_ref, we1t_ref, be1_ref,
              we2t_ref, be2_ref, w34_ref, wal_ref, bm1_ref, wm2_ref, bm2_ref,
              o_ref):
    blk = x_ref.shape[0]
    bk = blk * K

    gs = gs_ref[...]
    ak = gs[0:1, :]
    x0n = gs[1:2, :]
    x1n = gs[2:3, :]
    bnx = gs[3:4, :]
    bny = gs[4:5, :]
    cn = gs[5:6, :]
    sn = gs[6:7, :]
    lmn = gs[7:8, :]

    seg2 = seg2_ref[...]

    b5 = boxes_ref[...]
    th = b5[:, 4:5]
    d5 = jnp.concatenate(
        [b5[:, 0:2], jnp.cos(th), jnp.sin(th),
         jnp.log(jnp.minimum(b5[:, 2:3], b5[:, 3:4]))], axis=1)
    dp = lax.dot_general(d5, seg2, (((0,), (0,)), ((), ())),
                         preferred_element_type=jnp.float32)
    bix = dp[0:1, :]
    biy = dp[1:2, :]
    ci = dp[2:3, :]
    si = dp[3:4, :]
    lmi = dp[4:5, :]

    cth = ci * cn + si * sn
    sth = si * cn - ci * sn
    dx = bix - bnx
    dy = biy - bny
    dist = jnp.sqrt(dx * dx + dy * dy + 1e-12)
    scale = jnp.maximum(lmi - lmn, _LOG1EM6)
    u0 = (cth - 1.0) * x0n - sth * x1n
    u1 = sth * x0n + (cth - 1.0) * x1n

    e4 = jnp.concatenate([dist, scale, cth, sth], axis=0)
    h1t = jax.nn.relu(
        jnp.dot(we1t_ref[...], e4, preferred_element_type=jnp.float32)
        + be1_ref[...])
    e_embt = jax.nn.relu(
        jnp.dot(we2t_ref[...], h1t, preferred_element_type=jnp.float32)
        + be2_ref[...])

    l34 = jnp.concatenate([e_embt, u0, u1], axis=0)
    logits = ak + jnp.dot(wal_ref[...], l34,
                          preferred_element_type=jnp.float32)
    p = jnp.exp(logits)
    seg_p = seg2 * p

    m_hid = lax.dot_general(l34, w34_ref[...], (((0,), (0,)), ((), ())),
                            preferred_element_type=jnp.float32)
    h = jax.nn.relu(gy_ref[...] + m_hid + bm1_ref[...])

    agg = jnp.dot(seg_p, h, preferred_element_type=jnp.float32)
    den = jnp.sum(seg_p, axis=1, keepdims=True)
    agg = agg / den
    msg = jnp.dot(agg, wm2_ref[...],
                  preferred_element_type=jnp.float32) + bm2_ref[...]
    o_ref[...] = x_ref[...] + msg


def _message(gy, gs, boxes, x, seg2, we1t, be1, we2t, be2, w34, wal, bm1,
             wm2, bm2, *, interpret=False):
    blk = 80
    bk = blk * K
    return pl.pallas_call(
        _msg_body,
        grid=(N // blk,),
        in_specs=[
            pl.BlockSpec((bk, DY), lambda i: (i, 0)),
            pl.BlockSpec((8, bk), lambda i: (0, i)),
            pl.BlockSpec((blk, 5), lambda i: (i, 0)),
            pl.BlockSpec((blk, C), lambda i: (i, 0)),
            pl.BlockSpec((blk, bk), lambda i: (0, 0)),
            pl.BlockSpec((EH, 4), lambda i: (0, 0)),
            pl.BlockSpec((EH, 1), lambda i: (0, 0)),
            pl.BlockSpec((EH, EH), lambda i: (0, 0)),
            pl.BlockSpec((EH, 1), lambda i: (0, 0)),
            pl.BlockSpec((EH + 2, HID), lambda i: (0, 0)),
            pl.BlockSpec((1, EH + 2), lambda i: (0, 0)),
            pl.BlockSpec((1, HID), lambda i: (0, 0)),
            pl.BlockSpec((HID, C), lambda i: (0, 0)),
            pl.BlockSpec((1, C), lambda i: (0, 0)),
        ],
        out_specs=pl.BlockSpec((blk, C), lambda i: (i, 0)),
        out_shape=jax.ShapeDtypeStruct((N, C), jnp.float32),
        compiler_params=pltpu.CompilerParams(
            dimension_semantics=("arbitrary",)),
        interpret=interpret,
    )(gy, gs, boxes, x, seg2, we1t, be1, we2t, be2, w34, wal, bm1, wm2, bm2)


def kernel(x, nbr_idx, boxes, W_e1, b_e1, W_e2, b_e2, W_m1, b_m1, W_m2, b_m2,
           W_a, b_a):
    x = x.astype(jnp.float32)
    idx = nbr_idx.reshape(-1).astype(jnp.int32)

    wy = W_m1[:C, :]
    wak = W_a[C:2 * C, :]
    ty, ts = _pack_tables(x, boxes, wy, wak)

    gy, gs = _sc_gather(ty, ts, idx)

    blk3, bk3 = 80, 80 * K
    seg2 = (jnp.arange(bk3, dtype=jnp.int32) // K
            == jnp.arange(blk3, dtype=jnp.int32)[:, None]).astype(jnp.float32)
    w34 = jnp.concatenate([W_m1[C:, :], W_m1[0:2, :]], axis=0)
    wal = jnp.concatenate([W_a[2 * C:, :], W_a[C:C + 2, :]],
                          axis=0).T
    out = _message(gy, gs, boxes, x, seg2,
                   W_e1.T, b_e1.reshape(EH, 1), W_e2.T, b_e2.reshape(EH, 1),
                   w34, wal, b_m1.reshape(1, HID), W_m2, b_m2.reshape(1, C))
    return out
